# Initial kernel scaffold; baseline (speedup 1.0000x reference)
import jax
import jax.numpy as jnp
from jax.experimental import pallas as pl


def kernel(x, Wsel, bsel, Wexp, bexp):
    raise NotImplementedError("write your pallas kernel here")


# speedup vs baseline: 1.6519x; 1.6519x over previous
import functools

import jax
import jax.numpy as jnp
from jax.experimental import pallas as pl

EPS = 0.2


def _router_body(E, x_ref, wt_ref, b_ref, gate_ref, cost_ref):
    logits = jnp.dot(x_ref[...], wt_ref[...],
                     preferred_element_type=jnp.float32) + b_ref[...]
    m = jnp.max(logits, axis=1, keepdims=True)
    ex = jnp.exp(logits - m)
    w = ex / jnp.sum(ex, axis=1, keepdims=True)
    cols = jax.lax.broadcasted_iota(jnp.int32, w.shape, 1)
    prefix_cols = []
    rank_cols = []
    for j in range(E):
        wj = w[:, j:j + 1]
        ranked_above = (w > wj) | ((w == wj) & (cols < j))
        prefix_cols.append(
            jnp.sum(jnp.where(ranked_above, w, 0.0), axis=1, keepdims=True))
        rank_cols.append(
            jnp.sum(ranked_above.astype(jnp.int32), axis=1, keepdims=True))
    prefix = jnp.concatenate(prefix_cols, axis=1)
    rank = jnp.concatenate(rank_cols, axis=1)
    sw = jnp.maximum(0.0, jnp.minimum(w, (1.0 - EPS) - prefix))

    def perm_by_rank(v):
        outs = []
        for p in range(E):
            outs.append(jnp.sum(jnp.where(rank == p, v, 0.0), axis=1,
                                keepdims=True))
        return jnp.concatenate(outs, axis=1)

    gate_ref[...] = perm_by_rank(perm_by_rank(sw))
    active = sw > 0.0
    num_active = jnp.sum(active.astype(jnp.float32), axis=1, keepdims=True)
    min_active = jnp.min(jnp.where(active, sw, jnp.inf), axis=1,
                         keepdims=True)
    cost_ref[...] = num_active - 1.0 + min_active


def _expert_body(E, x_ref, w_ref, b_ref, g_ref, out_ref):
    e = pl.program_id(1)
    cols = jax.lax.broadcasted_iota(jnp.int32, g_ref.shape, 1)
    ge = jnp.sum(jnp.where(cols == e, g_ref[...], 0.0), axis=1, keepdims=True)
    y = jnp.dot(x_ref[...], w_ref[0],
                preferred_element_type=jnp.float32) + b_ref[0]
    contrib = ge * y

    @pl.when(e == 0)
    def _():
        out_ref[...] = contrib

    @pl.when(e > 0)
    def _():
        out_ref[...] += contrib


def kernel(x, Wsel, bsel, Wexp, bexp):
    N, D = x.shape
    E, _, OUT = Wexp.shape

    tb_r = min(1024, N)
    gate, cost = pl.pallas_call(
        functools.partial(_router_body, E),
        grid=(N // tb_r,),
        in_specs=[
            pl.BlockSpec((tb_r, D), lambda t: (t, 0)),
            pl.BlockSpec((D, E), lambda t: (0, 0)),
            pl.BlockSpec((1, E), lambda t: (0, 0)),
        ],
        out_specs=[
            pl.BlockSpec((tb_r, E), lambda t: (t, 0)),
            pl.BlockSpec((tb_r, 1), lambda t: (t, 0)),
        ],
        out_shape=[
            jax.ShapeDtypeStruct((N, E), jnp.float32),
            jax.ShapeDtypeStruct((N, 1), jnp.float32),
        ],
    )(x, Wsel.T, bsel.reshape(1, E))

    tb = min(1024, N)
    out = pl.pallas_call(
        functools.partial(_expert_body, E),
        grid=(N // tb, E),
        in_specs=[
            pl.BlockSpec((tb, D), lambda t, e: (t, 0)),
            pl.BlockSpec((1, D, OUT), lambda t, e: (e, 0, 0)),
            pl.BlockSpec((1, 1, OUT), lambda t, e: (e, 0, 0)),
            pl.BlockSpec((tb, E), lambda t, e: (t, 0)),
        ],
        out_specs=pl.BlockSpec((tb, OUT), lambda t, e: (t, 0)),
        out_shape=jax.ShapeDtypeStruct((N, OUT), jnp.float32),
    )(x, Wexp, bexp.reshape(E, 1, OUT), gate)

    return out, cost.reshape(N)
